# baseline (speedup 1.0000x reference)
import jax
import jax.numpy as jnp
from jax import lax
from jax.experimental import pallas as pl
from jax.experimental.pallas import tpu as pltpu
from jax.experimental.pallas import tpu_sc as plsc

_B, _N, _D, _C = 16, 4096, 1024, 8
_T = 32768

_NC, _NS, _L = 2, 16, 16
_NW = _NC * _NS
_TPW = _T // _NW
_GCHUNK = 128
_NBLK = _TPW // _GCHUNK
_GPB = _GCHUNK // _L


_NSTREAM = 4


def _proj_body(*refs):
    l_refs = refs[:_NSTREAM]
    wt_ref, b_ref, p_ref = refs[_NSTREAM:]
    for q in range(_NSTREAM):
        p_ref[q] = (
            lax.dot_general(l_refs[q][0], wt_ref[...],
                            (((1,), (1,)), ((), ())),
                            preferred_element_type=jnp.float32)
            + b_ref[...]
        )


def _project(latents_flat, WT, b2):
    M = latents_flat.shape[0]
    MQ = M // _NSTREAM
    BM = 1024
    l4 = latents_flat.reshape(_NSTREAM, MQ, _D)
    in_specs = [
        pl.BlockSpec((1, BM, _D), lambda i, q=q: (q, i, 0))
        for q in range(_NSTREAM)
    ]
    in_specs += [
        pl.BlockSpec((_C, _D), lambda i: (0, 0)),
        pl.BlockSpec((1, _C), lambda i: (0, 0)),
    ]
    p4 = pl.pallas_call(
        _proj_body,
        grid=(MQ // BM,),
        in_specs=in_specs,
        out_specs=pl.BlockSpec((_NSTREAM, BM, _C), lambda i: (0, i, 0)),
        out_shape=jax.ShapeDtypeStruct((_NSTREAM, MQ, _C), jnp.float32),
    )(*([l4] * _NSTREAM), WT, b2)
    return p4.reshape(M, _C)


def _sc_body(p_hbm, idx3_hbm, vals3_hbm, w_hbm,
             out3_hbm, loss_hbm, pres_hbm,
             idx_v, vals_v, w_v, gidx_v, rows_v,
             out_v, loss_v, pres_v,
             sem_i, sem_v, sem_w, sem_g, sem_o):
    cid = lax.axis_index("c")
    sid = lax.axis_index("s")
    wid = sid * _NC + cid
    base = wid * _TPW
    t0 = wid * _NBLK

    h_idx = pltpu.async_copy(idx3_hbm.at[pl.ds(t0, _NBLK)], idx_v, sem_i)
    h_vals = pltpu.async_copy(vals3_hbm.at[pl.ds(t0, _NBLK)], vals_v, sem_v)
    h_w = pltpu.async_copy(w_hbm.at[pl.ds(base, _TPW)], w_v, sem_w)

    pres_v[...] = jnp.zeros((_L,), jnp.int32)
    ones = jnp.ones((_L,), jnp.int32)
    h_idx.wait()

    copies = []
    for tb in range(_NBLK):
        for g in range(_GPB):
            sl = pl.ds(g * _L, _L)
            vb = idx_v[tb, 0, sl]
            vn = idx_v[tb, 1, sl]
            gidx_v[tb, sl] = vb * _N + vn
            plsc.store_scatter(pres_v, [vb], ones)
        copies.append(pltpu.async_copy(
            p_hbm.at[gidx_v.at[tb]],
            rows_v.at[pl.ds(tb * _GCHUNK, _GCHUNK)],
            sem_g,
        ))
    h_vals.wait()
    h_w.wait()

    acc = jnp.zeros((_L,), jnp.float32)
    out_copies = [None, None]
    for tb in range(_NBLK):
        copies[tb].wait()
        buf = tb % 2
        if out_copies[buf] is not None:
            out_copies[buf].wait()
        for g in range(_GPB):
            ids = lax.iota(jnp.int32, _L) + (tb * _GCHUNK + g * _L)
            sl = pl.ds(g * _L, _L)
            s = jnp.zeros((_L,), jnp.float32)
            for c in range(_C):
                cc = jnp.full((_L,), c, jnp.int32)
                rv = plsc.load_gather(rows_v, [ids, cc])
                out_v[buf, c, sl] = rv
                d = rv - vals_v[tb, c, sl]
                s = s + d * d
            acc = acc + w_v[pl.ds(tb * _GCHUNK + g * _L, _L)] * s
        out_copies[buf] = pltpu.async_copy(
            out_v.at[buf], out3_hbm.at[t0 + tb], sem_o)

    loss_v[...] = acc
    pltpu.sync_copy(loss_v, loss_hbm.at[wid])
    pltpu.sync_copy(pres_v, pres_hbm.at[wid])
    out_copies[0].wait()
    out_copies[1].wait()


def _sc_call(P, idx3, vals3, output_weights):
    mesh = plsc.VectorSubcoreMesh(
        core_axis_name="c", subcore_axis_name="s",
        num_cores=_NC, num_subcores=_NS,
    )
    f = pl.kernel(
        _sc_body,
        out_type=[
            jax.ShapeDtypeStruct((_T // _GCHUNK, _C, _GCHUNK), jnp.float32),
            jax.ShapeDtypeStruct((_NW, _L), jnp.float32),
            jax.ShapeDtypeStruct((_NW, _L), jnp.int32),
        ],
        mesh=mesh,
        compiler_params=pltpu.CompilerParams(
            needs_layout_passes=False, use_tc_tiling_on_sc=False),
        scratch_types=[
            pltpu.VMEM((_NBLK, 2, _GCHUNK), jnp.int32),
            pltpu.VMEM((_NBLK, _C, _GCHUNK), jnp.float32),
            pltpu.VMEM((_TPW,), jnp.float32),
            pltpu.VMEM((_NBLK, _GCHUNK), jnp.int32),
            pltpu.VMEM((_TPW, _C), jnp.float32),
            pltpu.VMEM((2, _C, _GCHUNK), jnp.float32),
            pltpu.VMEM((_L,), jnp.float32),
            pltpu.VMEM((_L,), jnp.int32),
            pltpu.SemaphoreType.DMA,
            pltpu.SemaphoreType.DMA,
            pltpu.SemaphoreType.DMA,
            pltpu.SemaphoreType.DMA,
            pltpu.SemaphoreType.DMA,
        ],
    )
    return f(P, idx3, vals3, output_weights)


def kernel(latents, output_task_indices, output_values, output_weights, W, b):
    latents_flat = latents.reshape(_B * _N, _D)
    P = _project(latents_flat, W.T, b.reshape(1, _C))
    idx3 = output_task_indices.reshape(_T // 128, 128, 2).transpose(0, 2, 1)
    vals3 = output_values.reshape(_T // 128, 128, _C).transpose(0, 2, 1)
    out3, loss_parts, pres = _sc_call(P, idx3, vals3, output_weights)
    out = out3.transpose(0, 2, 1).reshape(_T, _C)
    total = jnp.sum(loss_parts)
    nbatch = jnp.sum((jnp.sum(pres, axis=0) > 0).astype(jnp.float32))
    loss = total / (_T * _C) * nbatch / _B
    return out, loss


# speedup vs baseline: 1.0042x; 1.0042x over previous
import jax
import jax.numpy as jnp
from jax import lax
from jax.experimental import pallas as pl
from jax.experimental.pallas import tpu as pltpu
from jax.experimental.pallas import tpu_sc as plsc

_B, _N, _D, _C = 16, 4096, 1024, 8
_T = 32768

_NC, _NS, _L = 2, 16, 16
_NW = _NC * _NS
_TPW = _T // _NW
_GCHUNK = 128
_NBLK = _TPW // _GCHUNK
_GPB = _GCHUNK // _L


def _proj_body(l_ref, wt_ref, b_ref, p_ref):
    p_ref[...] = (
        lax.dot_general(l_ref[...], wt_ref[...], (((1,), (1,)), ((), ())),
                        preferred_element_type=jnp.float32)
        + b_ref[...]
    )


def _project(latents_flat, WT, b2):
    M = latents_flat.shape[0]
    BM = 2048
    return pl.pallas_call(
        _proj_body,
        grid=(M // BM,),
        in_specs=[
            pl.BlockSpec((BM, _D), lambda i: (i, 0)),
            pl.BlockSpec((_C, _D), lambda i: (0, 0)),
            pl.BlockSpec((1, _C), lambda i: (0, 0)),
        ],
        out_specs=pl.BlockSpec((BM, _C), lambda i: (i, 0)),
        out_shape=jax.ShapeDtypeStruct((M, _C), jnp.float32),
    )(latents_flat, WT, b2)


def _sc_body(p_hbm, idx3_hbm, vals3_hbm, w_hbm,
             out3_hbm, loss_hbm, pres_hbm,
             idx_v, vals_v, w_v, gidx_v, rows_v,
             out_v, loss_v, pres_v,
             sem_i, sem_v, sem_w, sem_g, sem_o):
    cid = lax.axis_index("c")
    sid = lax.axis_index("s")
    wid = sid * _NC + cid
    base = wid * _TPW
    t0 = wid * _NBLK

    h_idx = pltpu.async_copy(idx3_hbm.at[pl.ds(t0, _NBLK)], idx_v, sem_i)
    h_vals = pltpu.async_copy(vals3_hbm.at[pl.ds(t0, _NBLK)], vals_v, sem_v)
    h_w = pltpu.async_copy(w_hbm.at[pl.ds(base, _TPW)], w_v, sem_w)

    pres_v[...] = jnp.zeros((_L,), jnp.int32)
    ones = jnp.ones((_L,), jnp.int32)
    h_idx.wait()

    copies = []
    for tb in range(_NBLK):
        for g in range(_GPB):
            sl = pl.ds(g * _L, _L)
            vb = idx_v[tb, 0, sl]
            vn = idx_v[tb, 1, sl]
            gidx_v[tb, sl] = vb * _N + vn
            plsc.store_scatter(pres_v, [vb], ones)
        copies.append(pltpu.async_copy(
            p_hbm.at[gidx_v.at[tb]],
            rows_v.at[pl.ds(tb * _GCHUNK, _GCHUNK)],
            sem_g,
        ))
    h_vals.wait()
    h_w.wait()

    acc = jnp.zeros((_L,), jnp.float32)
    out_copies = [None, None]
    for tb in range(_NBLK):
        copies[tb].wait()
        buf = tb % 2
        if out_copies[buf] is not None:
            out_copies[buf].wait()
        for g in range(_GPB):
            ids = lax.iota(jnp.int32, _L) + (tb * _GCHUNK + g * _L)
            sl = pl.ds(g * _L, _L)
            s = jnp.zeros((_L,), jnp.float32)
            for c in range(_C):
                cc = jnp.full((_L,), c, jnp.int32)
                rv = plsc.load_gather(rows_v, [ids, cc])
                out_v[buf, c, sl] = rv
                d = rv - vals_v[tb, c, sl]
                s = s + d * d
            acc = acc + w_v[pl.ds(tb * _GCHUNK + g * _L, _L)] * s
        out_copies[buf] = pltpu.async_copy(
            out_v.at[buf], out3_hbm.at[t0 + tb], sem_o)

    loss_v[...] = acc
    pltpu.sync_copy(loss_v, loss_hbm.at[wid])
    pltpu.sync_copy(pres_v, pres_hbm.at[wid])
    out_copies[0].wait()
    out_copies[1].wait()


def _sc_call(P, idx3, vals3, output_weights):
    mesh = plsc.VectorSubcoreMesh(
        core_axis_name="c", subcore_axis_name="s",
        num_cores=_NC, num_subcores=_NS,
    )
    f = pl.kernel(
        _sc_body,
        out_type=[
            jax.ShapeDtypeStruct((_T // _GCHUNK, _C, _GCHUNK), jnp.float32),
            jax.ShapeDtypeStruct((_NW, _L), jnp.float32),
            jax.ShapeDtypeStruct((_NW, _L), jnp.int32),
        ],
        mesh=mesh,
        compiler_params=pltpu.CompilerParams(
            needs_layout_passes=False, use_tc_tiling_on_sc=False),
        scratch_types=[
            pltpu.VMEM((_NBLK, 2, _GCHUNK), jnp.int32),
            pltpu.VMEM((_NBLK, _C, _GCHUNK), jnp.float32),
            pltpu.VMEM((_TPW,), jnp.float32),
            pltpu.VMEM((_NBLK, _GCHUNK), jnp.int32),
            pltpu.VMEM((_TPW, _C), jnp.float32),
            pltpu.VMEM((2, _C, _GCHUNK), jnp.float32),
            pltpu.VMEM((_L,), jnp.float32),
            pltpu.VMEM((_L,), jnp.int32),
            pltpu.SemaphoreType.DMA,
            pltpu.SemaphoreType.DMA,
            pltpu.SemaphoreType.DMA,
            pltpu.SemaphoreType.DMA,
            pltpu.SemaphoreType.DMA,
        ],
    )
    return f(P, idx3, vals3, output_weights)


def kernel(latents, output_task_indices, output_values, output_weights, W, b):
    latents_flat = latents.reshape(_B * _N, _D)
    P = _project(latents_flat, W.T, b.reshape(1, _C))
    idx3 = output_task_indices.reshape(_T // 128, 128, 2).transpose(0, 2, 1)
    vals3 = output_values.reshape(_T // 128, 128, _C).transpose(0, 2, 1)
    out3, loss_parts, pres = _sc_call(P, idx3, vals3, output_weights)
    out = out3.transpose(0, 2, 1).reshape(_T, _C)
    total = jnp.sum(loss_parts)
    nbatch = jnp.sum((jnp.sum(pres, axis=0) > 0).astype(jnp.float32))
    loss = total / (_T * _C) * nbatch / _B
    return out, loss
